# baseline (speedup 1.0000x reference)
import functools

import jax
import jax.numpy as jnp
from jax import lax
from jax.experimental import pallas as pl
from jax.experimental.pallas import tpu as pltpu
from jax.experimental.pallas import tpu_sc as plsc

_NC = 2
_NS = 16
_NW = _NC * _NS

_CHUNK = 128
_NBUF = 8


@functools.lru_cache(maxsize=None)
def _build(chunks_per_worker, d_model):
  mesh = plsc.VectorSubcoreMesh(core_axis_name="c", subcore_axis_name="s")
  nbuf = _NBUF
  blocks = chunks_per_worker // nbuf

  @functools.partial(
      pl.kernel,
      out_type=jax.ShapeDtypeStruct(
          (_NW, chunks_per_worker, _CHUNK, d_model), jnp.float32),
      mesh=mesh,
      scratch_types=[
          pltpu.VMEM((chunks_per_worker, _CHUNK), jnp.int32),
          pltpu.VMEM((nbuf, _CHUNK, d_model), jnp.float32),
      ] + [pltpu.SemaphoreType.DMA] * nbuf,
      compiler_params=pltpu.CompilerParams(use_tc_tiling_on_sc=False),
  )
  def gather_kernel(idx_hbm, table_hbm, out_hbm, idx_v, rows_v, *sems):
    wid = lax.axis_index("s") * _NC + lax.axis_index("c")
    pltpu.sync_copy(idx_hbm.at[wid], idx_v)

    def start(chunk, buf):
      pltpu.async_copy(table_hbm.at[idx_v.at[chunk]], rows_v.at[buf],
                       sems[buf])

    def finish(chunk, buf):
      pltpu.make_async_copy(table_hbm.at[idx_v.at[chunk]], rows_v.at[buf],
                            sems[buf]).wait()
      pltpu.sync_copy(rows_v.at[buf], out_hbm.at[wid, chunk])

    for b in range(nbuf):
      start(b, b)

    @pl.loop(0, blocks - 1)
    def _(t):
      base = t * nbuf
      for b in range(nbuf):
        finish(base + b, b)
        start(base + b + nbuf, b)

    tail = (blocks - 1) * nbuf
    for b in range(nbuf):
      finish(tail + b, b)

  return gather_kernel


def kernel(token_ids, W):
  orig_shape = token_ids.shape
  d_model = W.shape[1]
  idx = token_ids.reshape(-1).astype(jnp.int32)
  n = idx.shape[0]

  per_worker_chunk = _CHUNK * _NW * _NBUF
  pad = (-n) % per_worker_chunk
  if pad:
    idx = jnp.concatenate([idx, jnp.zeros((pad,), jnp.int32)])
  total = n + pad
  chunks_per_worker = total // (_NW * _CHUNK)

  idx = idx.reshape(_NW, chunks_per_worker, _CHUNK)
  out = _build(chunks_per_worker, d_model)(idx, W)
  out = out.reshape(total, d_model)
  if pad:
    out = out[:n]
  return out.reshape(*orig_shape, d_model)


# speedup vs baseline: 1.0008x; 1.0008x over previous
import functools

import jax
import jax.numpy as jnp
from jax import lax
from jax.experimental import pallas as pl
from jax.experimental.pallas import tpu as pltpu
from jax.experimental.pallas import tpu_sc as plsc

_NC = 2
_NS = 16
_NW = _NC * _NS

_MAX_GATHER = 128
_NBUF = 4


@functools.lru_cache(maxsize=None)
def _build(batch, hist, d_model):
  assert batch % _NW == 0 and hist % 8 == 0
  rows_per_worker = batch // _NW
  nbuf = _NBUF
  assert rows_per_worker % nbuf == 0
  blocks = rows_per_worker // nbuf
  chunks = [(o, min(_MAX_GATHER, hist - o)) for o in range(0, hist, _MAX_GATHER)]
  mesh = plsc.VectorSubcoreMesh(core_axis_name="c", subcore_axis_name="s")

  @functools.partial(
      pl.kernel,
      out_type=jax.ShapeDtypeStruct((batch, hist, d_model), jnp.float32),
      mesh=mesh,
      scratch_types=[
          pltpu.VMEM((rows_per_worker, hist), jnp.int32),
          pltpu.VMEM((nbuf, hist, d_model), jnp.float32),
      ] + [pltpu.SemaphoreType.DMA] * nbuf,
      compiler_params=pltpu.CompilerParams(use_tc_tiling_on_sc=False),
  )
  def gather_kernel(idx_hbm, table_hbm, out_hbm, idx_v, rows_v, *sems):
    wid = lax.axis_index("s") * _NC + lax.axis_index("c")
    row0 = wid * rows_per_worker
    pltpu.sync_copy(idx_hbm.at[pl.ds(row0, rows_per_worker)], idx_v)

    def start(r, b):
      for o, n in chunks:
        pltpu.async_copy(table_hbm.at[idx_v.at[r, pl.ds(o, n)]],
                         rows_v.at[b, pl.ds(o, n)], sems[b])

    def finish(r, b):
      for o, n in chunks:
        pltpu.make_async_copy(table_hbm.at[idx_v.at[r, pl.ds(o, n)]],
                              rows_v.at[b, pl.ds(o, n)], sems[b]).wait()
      pltpu.sync_copy(rows_v.at[b], out_hbm.at[row0 + r])

    for b in range(nbuf):
      start(b, b)

    @pl.loop(0, blocks - 1)
    def _(t):
      base = t * nbuf
      for b in range(nbuf):
        finish(base + b, b)
        start(base + b + nbuf, b)

    tail = (blocks - 1) * nbuf
    for b in range(nbuf):
      finish(tail + b, b)

  return gather_kernel


def kernel(token_ids, W):
  batch, hist = token_ids.shape
  return _build(batch, hist, W.shape[1])(token_ids.astype(jnp.int32), W)
